# baseline (speedup 1.0000x reference)
import functools

import jax
import jax.numpy as jnp
from jax import lax
from jax.experimental import pallas as pl
from jax.experimental.pallas import tpu as pltpu
from jax.experimental.pallas import tpu_sc as plsc

H = 16
BUCKET = 4
EPS = 1e-4


def _proj_body(x_ref, wqv_ref, bqv_ref, wh_ref, qv_ref, ang_ref):
    x = x_ref[0]
    mm = jnp.dot(x, wqv_ref[...], preferred_element_type=jnp.float32) + bqv_ref[...]
    for h in range(H):
        qv_ref[h] = mm[:, 128 * h:128 * (h + 1)]
    hsh = jnp.dot(x, wh_ref[...], preferred_element_type=jnp.float32)
    ang_ref[...] = hsh[:, :H] / (hsh[:, H:] + EPS)


def _projections(x, W_qv, b_qv, Wh2, b):
    B, S, D = x.shape
    Sb = 512
    grid = (S // Sb,)
    return pl.pallas_call(
        _proj_body,
        grid=grid,
        in_specs=[
            pl.BlockSpec((1, Sb, D), lambda s: (b, s, 0)),
            pl.BlockSpec((D, 2 * D), lambda s: (0, 0)),
            pl.BlockSpec((1, 2 * D), lambda s: (0, 0)),
            pl.BlockSpec((D, 2 * H), lambda s: (0, 0)),
        ],
        out_specs=[
            pl.BlockSpec((H, Sb, 128), lambda s: (0, s, 0)),
            pl.BlockSpec((Sb, H), lambda s: (s, 0)),
        ],
        out_shape=[
            jax.ShapeDtypeStruct((H, S, 128), jnp.float32),
            jax.ShapeDtypeStruct((S, H), jnp.float32),
        ],
    )(x, W_qv, b_qv, Wh2)


def _ce_stage(keys, payload, g, d, k, CS=1024):
    bit_d = (g & d) != 0
    swapm = bit_d ^ ((g & k) != 0)
    if d < CS:
        pk = jnp.where(bit_d, jnp.roll(keys, d, axis=0), jnp.roll(keys, -d, axis=0))
        pp = jnp.where(bit_d, jnp.roll(payload, d, axis=0), jnp.roll(payload, -d, axis=0))
    else:
        dl = (d // CS) * 16
        pk = jnp.where(bit_d, jnp.roll(keys, dl, axis=1), jnp.roll(keys, -dl, axis=1))
        pp = jnp.where(bit_d, jnp.roll(payload, dl, axis=1), jnp.roll(payload, -dl, axis=1))
    cmp = (pk < keys) | ((pk == keys) & (pp < payload))
    take = cmp ^ swapm
    return jnp.where(take, pk, keys), jnp.where(take, pp, payload)


def _sort_body(keys_ref, out_ref, *, S):
    keys = keys_ref[...]
    sub = lax.broadcasted_iota(jnp.int32, keys.shape, 0)
    lane = lax.broadcasted_iota(jnp.int32, keys.shape, 1)
    g = (lane >> 4) * 1024 + sub
    payload = g
    kk = 2
    while kk <= S:
        d = kk // 2
        while d >= 1:
            keys, payload = _ce_stage(keys, payload, g, d, kk)
            d //= 2
        kk *= 2
    out_ref[...] = (lane & (H - 1)) * S + payload


def _bitonic_sort(keys, S):
    return pl.pallas_call(
        functools.partial(_sort_body, S=S),
        grid=(1,),
        in_specs=[pl.BlockSpec((1024, 128), lambda i: (0, 0))],
        out_specs=pl.BlockSpec((1024, 128), lambda i: (0, 0)),
        out_shape=jax.ShapeDtypeStruct((1024, 128), jnp.int32),
    )(keys)


def _sc_gather(qv_flat, sidx):
    NP, NJ = sidx.shape[0], sidx.shape[1]
    S = NJ * 128
    info = plsc.get_sparse_core_info()
    NC = info.num_cores
    mesh = plsc.VectorSubcoreMesh(core_axis_name="c", subcore_axis_name="s")

    @functools.partial(
        pl.kernel, mesh=mesh,
        out_type=jax.ShapeDtypeStruct((NP, S, 128), jnp.float32),
        scratch_types=[
            pltpu.VMEM((NJ // 2, 128), jnp.int32),
            pltpu.VMEM((128, 128), jnp.float32),
            pltpu.SemaphoreType.DMA,
        ],
    )
    def k(qv_hbm, sidx_hbm, out_hbm, idx_v, buf, sem):
        wid = lax.axis_index("s") * NC + lax.axis_index("c")
        p = wid >> 1
        jbase = (wid & 1) * (NJ // 2)
        pltpu.sync_copy(sidx_hbm.at[p, pl.ds(jbase * 1, NJ // 2)], idx_v)

        def body(j, carry):
            pltpu.async_copy(qv_hbm.at[idx_v.at[j]], buf, sem).wait()
            pltpu.sync_copy(buf, out_hbm.at[p, pl.ds((jbase + j) * 128, 128)])
            return carry

        lax.fori_loop(0, NJ // 2, body, 0)

    return k(qv_flat, sidx)


def _sc_scatter(os_, sidx):
    NP, NJ = sidx.shape[0], sidx.shape[1]
    S = NJ * 128
    info = plsc.get_sparse_core_info()
    NC = info.num_cores
    mesh = plsc.VectorSubcoreMesh(core_axis_name="c", subcore_axis_name="s")

    @functools.partial(
        pl.kernel, mesh=mesh,
        out_type=jax.ShapeDtypeStruct((NP * S, 128), jnp.float32),
        scratch_types=[
            pltpu.VMEM((NJ // 2, 128), jnp.int32),
            pltpu.VMEM((128, 128), jnp.float32),
            pltpu.SemaphoreType.DMA,
        ],
    )
    def k(os_hbm, sidx_hbm, out_hbm, idx_v, buf, sem):
        wid = lax.axis_index("s") * NC + lax.axis_index("c")
        p = wid >> 1
        jbase = (wid & 1) * (NJ // 2)
        pltpu.sync_copy(sidx_hbm.at[p, pl.ds(jbase * 1, NJ // 2)], idx_v)

        def body(j, carry):
            pltpu.sync_copy(os_hbm.at[p, pl.ds((jbase + j) * 128, 128)], buf)
            pltpu.async_copy(buf, out_hbm.at[idx_v.at[j]], sem).wait()
            return carry

        lax.fori_loop(0, NJ // 2, body, 0)

    return k(os_, sidx)


def _attn_body(qv_ref, o_ref, *, Ts, S):
    qv = qv_ref[0]
    bi = lax.broadcasted_iota(jnp.int32, (Ts, Ts), 0) // BUCKET
    bj = lax.broadcasted_iota(jnp.int32, (Ts, Ts), 1) // BUCKET
    mask01 = jnp.where(bi == bj, 1.0, 0.0)
    for t in range(S // Ts):
        blk = qv[Ts * t:Ts * (t + 1)]
        q = blk[:, :64]
        v = blk[:, 64:]
        s = lax.dot_general(q, q, (((1,), (1,)), ((), ())),
                            preferred_element_type=jnp.float32)
        e = jnp.exp(s * 0.125) * mask01
        p = e / jnp.sum(e, axis=-1, keepdims=True)
        o = jnp.dot(p, v, preferred_element_type=jnp.float32)
        o_ref[0, Ts * t:Ts * (t + 1), :64] = o
        o_ref[0, Ts * t:Ts * (t + 1), 64:] = jnp.zeros_like(o)


def _bucket_attention(qvs):
    NP, S, _ = qvs.shape
    Ts = 128
    grid = (NP,)
    return pl.pallas_call(
        functools.partial(_attn_body, Ts=Ts, S=S),
        grid=grid,
        in_specs=[pl.BlockSpec((1, S, 128), lambda g: (g, 0, 0))],
        out_specs=pl.BlockSpec((1, S, 128), lambda g: (g, 0, 0)),
        out_shape=jax.ShapeDtypeStruct((NP, S, 128), jnp.float32),
    )(qvs)


def _outproj_body(o4_ref, wo_ref, bo_ref, out_ref):
    acc = bo_ref[...].astype(jnp.float32)
    for kgrp in range(4):
        blk4 = jnp.concatenate(
            [o4_ref[4 * kgrp + j, :, :64] for j in range(4)], axis=1)
        acc = acc + jnp.dot(blk4, wo_ref[256 * kgrp:256 * (kgrp + 1), :],
                            preferred_element_type=jnp.float32)
    out_ref[0] = acc


def _outproj_body_alias(o4_ref, wo_ref, bo_ref, prev_ref, out_ref):
    del prev_ref
    _outproj_body(o4_ref, wo_ref, bo_ref, out_ref)


def _out_projection(o4, W_o, b_o, b, B, prev=None):
    _, S, _ = o4.shape
    D = W_o.shape[0]
    Sb = 512
    grid = (S // Sb,)
    in_specs = [
        pl.BlockSpec((H, Sb, 128), lambda s: (0, s, 0)),
        pl.BlockSpec((D, D), lambda s: (0, 0)),
        pl.BlockSpec((1, D), lambda s: (0, 0)),
    ]
    args = [o4, W_o, b_o.reshape(1, D)]
    body = _outproj_body
    kwargs = {}
    if prev is not None:
        in_specs.append(pl.BlockSpec(memory_space=pl.ANY))
        args.append(prev)
        body = _outproj_body_alias
        kwargs = dict(input_output_aliases={3: 0})
    return pl.pallas_call(
        body,
        grid=grid,
        in_specs=in_specs,
        out_specs=pl.BlockSpec((1, Sb, D), lambda s: (b, s, 0)),
        out_shape=jax.ShapeDtypeStruct((B, S, D), jnp.float32),
        **kwargs,
    )(*args)


def kernel(x, W_hash, W_q, b_q, W_v, b_v, W_o, b_o):
    B, S, D = x.shape
    dh = D // H
    W_qv = jnp.concatenate(
        [W_q.reshape(D, H, dh), W_v.reshape(D, H, dh)], axis=2).reshape(D, 2 * D)
    b_qv = jnp.concatenate(
        [b_q.reshape(H, dh), b_v.reshape(H, dh)], axis=1).reshape(1, 2 * D)
    Wh2 = W_hash.reshape(D, H, 2).transpose(0, 2, 1).reshape(D, 2 * H)

    def pack_keys(ang_b):
        return ang_b.reshape(8, 1024, H).transpose(1, 0, 2).reshape(1024, 128)

    def unpack_sidx(sidxp):
        return (sidxp.reshape(1024, 8, H).transpose(2, 1, 0)
                .reshape(H, S // 128, 128))

    qv0, ang0 = _projections(x, W_qv, b_qv, Wh2, 0)
    sidx0 = unpack_sidx(_bitonic_sort(pack_keys(ang0), S))
    qvs0 = _sc_gather(qv0.reshape(H * S, 128), sidx0)
    qv1, ang1 = _projections(x, W_qv, b_qv, Wh2, 1)
    sidx1 = unpack_sidx(_bitonic_sort(pack_keys(ang1), S))
    os0 = _bucket_attention(qvs0)
    qvs1 = _sc_gather(qv1.reshape(H * S, 128), sidx1)
    o40 = _sc_scatter(os0, sidx0).reshape(H, S, 128)
    os1 = _bucket_attention(qvs1)
    out0 = _out_projection(o40, W_o, b_o, 0, B)
    o41 = _sc_scatter(os1, sidx1).reshape(H, S, 128)
    return _out_projection(o41, W_o, b_o, 1, B, prev=out0)


# speedup vs baseline: 1.4720x; 1.4720x over previous
import functools

import jax
import jax.numpy as jnp
from jax import lax
from jax.experimental import pallas as pl
from jax.experimental.pallas import tpu as pltpu
from jax.experimental.pallas import tpu_sc as plsc

H = 16
BUCKET = 4
EPS = 1e-4


def _proj_body(x_ref, wqv_ref, bqv_ref, wh_ref, qv_ref, ang_ref):
    x = x_ref[0]
    mm = jnp.dot(x, wqv_ref[...], preferred_element_type=jnp.float32) + bqv_ref[...]
    for h in range(H):
        qv_ref[h] = mm[:, 128 * h:128 * (h + 1)]
    hsh = jnp.dot(x, wh_ref[...], preferred_element_type=jnp.float32)
    ang_ref[...] = hsh[:, :H] / (hsh[:, H:] + EPS)


def _projections(x, W_qv, b_qv, Wh2, b):
    B, S, D = x.shape
    Sb = 512
    grid = (S // Sb,)
    return pl.pallas_call(
        _proj_body,
        grid=grid,
        in_specs=[
            pl.BlockSpec((1, Sb, D), lambda s: (b, s, 0)),
            pl.BlockSpec((D, 2 * D), lambda s: (0, 0)),
            pl.BlockSpec((1, 2 * D), lambda s: (0, 0)),
            pl.BlockSpec((D, 2 * H), lambda s: (0, 0)),
        ],
        out_specs=[
            pl.BlockSpec((H, Sb, 128), lambda s: (0, s, 0)),
            pl.BlockSpec((Sb, H), lambda s: (s, 0)),
        ],
        out_shape=[
            jax.ShapeDtypeStruct((H, S, 128), jnp.float32),
            jax.ShapeDtypeStruct((S, H), jnp.float32),
        ],
    )(x, W_qv, b_qv, Wh2)


def _ce_stage(keys, payload, g, d, k, CS=1024):
    bit_d = (g & d) != 0
    swapm = bit_d ^ ((g & k) != 0)
    if d < CS:
        pk = jnp.where(bit_d, jnp.roll(keys, d, axis=0), jnp.roll(keys, -d, axis=0))
        pp = jnp.where(bit_d, jnp.roll(payload, d, axis=0), jnp.roll(payload, -d, axis=0))
    else:
        dl = (d // CS) * 16
        pk = jnp.where(bit_d, jnp.roll(keys, dl, axis=1), jnp.roll(keys, -dl, axis=1))
        pp = jnp.where(bit_d, jnp.roll(payload, dl, axis=1), jnp.roll(payload, -dl, axis=1))
    cmp = (pk < keys) | ((pk == keys) & (pp < payload))
    take = cmp ^ swapm
    return jnp.where(take, pk, keys), jnp.where(take, pp, payload)


def _sort_body(keys_ref, out_ref, *, S):
    keys = keys_ref[...]
    sub = lax.broadcasted_iota(jnp.int32, keys.shape, 0)
    lane = lax.broadcasted_iota(jnp.int32, keys.shape, 1)
    g = (lane >> 4) * 1024 + sub
    payload = g
    kk = 2
    while kk <= S:
        d = kk // 2
        while d >= 1:
            keys, payload = _ce_stage(keys, payload, g, d, kk)
            d //= 2
        kk *= 2
    out_ref[...] = (lane & (H - 1)) * S + payload


def _bitonic_sort(keys, S):
    return pl.pallas_call(
        functools.partial(_sort_body, S=S),
        grid=(1,),
        in_specs=[pl.BlockSpec((1024, 128), lambda i: (0, 0))],
        out_specs=pl.BlockSpec((1024, 128), lambda i: (0, 0)),
        out_shape=jax.ShapeDtypeStruct((1024, 128), jnp.int32),
    )(keys)


def _sc_gather(qv_flat, sidx):
    NP, NJ = sidx.shape[0], sidx.shape[1]
    S = NJ * 128
    info = plsc.get_sparse_core_info()
    NC = info.num_cores
    mesh = plsc.VectorSubcoreMesh(core_axis_name="c", subcore_axis_name="s")

    @functools.partial(
        pl.kernel, mesh=mesh,
        out_type=jax.ShapeDtypeStruct((NP, S, 128), jnp.float32),
        scratch_types=[
            pltpu.VMEM((NJ // 2, 128), jnp.int32),
            pltpu.VMEM((128, 128), jnp.float32),
            pltpu.SemaphoreType.DMA,
        ],
    )
    def k(qv_hbm, sidx_hbm, out_hbm, idx_v, buf, sem):
        wid = lax.axis_index("s") * NC + lax.axis_index("c")
        p = wid >> 1
        jbase = (wid & 1) * (NJ // 2)
        pltpu.sync_copy(sidx_hbm.at[p, pl.ds(jbase * 1, NJ // 2)], idx_v)

        def body(j, carry):
            pltpu.async_copy(qv_hbm.at[idx_v.at[j]], buf, sem).wait()
            pltpu.sync_copy(buf, out_hbm.at[p, pl.ds((jbase + j) * 128, 128)])
            return carry

        lax.fori_loop(0, NJ // 2, body, 0)

    return k(qv_flat, sidx)


def _sc_scatter(os_, sidx):
    NP, NJ = sidx.shape[0], sidx.shape[1]
    S = NJ * 128
    info = plsc.get_sparse_core_info()
    NC = info.num_cores
    mesh = plsc.VectorSubcoreMesh(core_axis_name="c", subcore_axis_name="s")

    @functools.partial(
        pl.kernel, mesh=mesh,
        out_type=jax.ShapeDtypeStruct((NP * S, 128), jnp.float32),
        scratch_types=[
            pltpu.VMEM((NJ // 2, 128), jnp.int32),
            pltpu.VMEM((128, 128), jnp.float32),
            pltpu.SemaphoreType.DMA,
        ],
    )
    def k(os_hbm, sidx_hbm, out_hbm, idx_v, buf, sem):
        wid = lax.axis_index("s") * NC + lax.axis_index("c")
        p = wid >> 1
        jbase = (wid & 1) * (NJ // 2)
        pltpu.sync_copy(sidx_hbm.at[p, pl.ds(jbase * 1, NJ // 2)], idx_v)

        def body(j, carry):
            pltpu.sync_copy(os_hbm.at[p, pl.ds((jbase + j) * 128, 128)], buf)
            pltpu.async_copy(buf, out_hbm.at[idx_v.at[j]], sem).wait()
            return carry

        lax.fori_loop(0, NJ // 2, body, 0)

    return k(os_, sidx)


def _attn_body(qv_ref, o_ref, *, Ts, S):
    qv = qv_ref[0]
    bi = lax.broadcasted_iota(jnp.int32, (Ts, Ts), 0) // BUCKET
    bj = lax.broadcasted_iota(jnp.int32, (Ts, Ts), 1) // BUCKET
    mask01 = jnp.where(bi == bj, 1.0, 0.0)
    for t in range(S // Ts):
        blk = qv[Ts * t:Ts * (t + 1)]
        q = blk[:, :64]
        v = blk[:, 64:]
        s = lax.dot_general(q, q, (((1,), (1,)), ((), ())),
                            preferred_element_type=jnp.float32)
        e = jnp.exp(s * 0.125) * mask01
        p = e / jnp.sum(e, axis=-1, keepdims=True)
        o = jnp.dot(p, v, preferred_element_type=jnp.float32)
        o_ref[0, Ts * t:Ts * (t + 1), :64] = o
        o_ref[0, Ts * t:Ts * (t + 1), 64:] = jnp.zeros_like(o)


def _bucket_attention(qvs):
    NP, S, _ = qvs.shape
    Ts = 256
    grid = (NP,)
    return pl.pallas_call(
        functools.partial(_attn_body, Ts=Ts, S=S),
        grid=grid,
        in_specs=[pl.BlockSpec((1, S, 128), lambda g: (g, 0, 0))],
        out_specs=pl.BlockSpec((1, S, 128), lambda g: (g, 0, 0)),
        out_shape=jax.ShapeDtypeStruct((NP, S, 128), jnp.float32),
    )(qvs)


def _outproj_body(o4_ref, wo_ref, bo_ref, out_ref):
    acc = bo_ref[...].astype(jnp.float32)
    for kgrp in range(4):
        blk4 = jnp.concatenate(
            [o4_ref[4 * kgrp + j, :, :64] for j in range(4)], axis=1)
        acc = acc + jnp.dot(blk4, wo_ref[256 * kgrp:256 * (kgrp + 1), :],
                            preferred_element_type=jnp.float32)
    out_ref[0] = acc


def _outproj_body_alias(o4_ref, wo_ref, bo_ref, prev_ref, out_ref):
    del prev_ref
    _outproj_body(o4_ref, wo_ref, bo_ref, out_ref)


def _out_projection(o4, W_o, b_o, b, B, prev=None):
    _, S, _ = o4.shape
    D = W_o.shape[0]
    Sb = 512
    grid = (S // Sb,)
    in_specs = [
        pl.BlockSpec((H, Sb, 128), lambda s: (0, s, 0)),
        pl.BlockSpec((D, D), lambda s: (0, 0)),
        pl.BlockSpec((1, D), lambda s: (0, 0)),
    ]
    args = [o4, W_o, b_o.reshape(1, D)]
    body = _outproj_body
    kwargs = {}
    if prev is not None:
        in_specs.append(pl.BlockSpec(memory_space=pl.ANY))
        args.append(prev)
        body = _outproj_body_alias
        kwargs = dict(input_output_aliases={3: 0})
    return pl.pallas_call(
        body,
        grid=grid,
        in_specs=in_specs,
        out_specs=pl.BlockSpec((1, Sb, D), lambda s: (b, s, 0)),
        out_shape=jax.ShapeDtypeStruct((B, S, D), jnp.float32),
        **kwargs,
    )(*args)


def kernel(x, W_hash, W_q, b_q, W_v, b_v, W_o, b_o):
    B, S, D = x.shape
    dh = D // H
    W_qv = jnp.concatenate(
        [W_q.reshape(D, H, dh), W_v.reshape(D, H, dh)], axis=2).reshape(D, 2 * D)
    b_qv = jnp.concatenate(
        [b_q.reshape(H, dh), b_v.reshape(H, dh)], axis=1).reshape(1, 2 * D)
    Wh2 = W_hash.reshape(D, H, 2).transpose(0, 2, 1).reshape(D, 2 * H)

    def pack_keys(ang_b):
        return ang_b.reshape(8, 1024, H).transpose(1, 0, 2).reshape(1024, 128)

    def unpack_sidx(sidxp):
        return (sidxp.reshape(1024, 8, H).transpose(2, 1, 0)
                .reshape(H, S // 128, 128))

    qv0, ang0 = _projections(x, W_qv, b_qv, Wh2, 0)
    sidx0 = unpack_sidx(_bitonic_sort(pack_keys(ang0), S))
    qvs0 = _sc_gather(qv0.reshape(H * S, 128), sidx0)
    qv1, ang1 = _projections(x, W_qv, b_qv, Wh2, 1)
    sidx1 = unpack_sidx(_bitonic_sort(pack_keys(ang1), S))
    os0 = _bucket_attention(qvs0)
    qvs1 = _sc_gather(qv1.reshape(H * S, 128), sidx1)
    o40 = _sc_scatter(os0, sidx0).reshape(H, S, 128)
    os1 = _bucket_attention(qvs1)
    out0 = _out_projection(o40, W_o, b_o, 0, B)
    o41 = _sc_scatter(os1, sidx1).reshape(H, S, 128)
    return _out_projection(o41, W_o, b_o, 1, B, prev=out0)


# speedup vs baseline: 1.5050x; 1.0224x over previous
import functools

import jax
import jax.numpy as jnp
from jax import lax
from jax.experimental import pallas as pl
from jax.experimental.pallas import tpu as pltpu
from jax.experimental.pallas import tpu_sc as plsc

H = 16
BUCKET = 4
EPS = 1e-4


def _proj_body(x_ref, wqv_ref, bqv_ref, wh_ref, qv_ref, ang_ref):
    x = x_ref[0]
    mm = jnp.dot(x, wqv_ref[...], preferred_element_type=jnp.float32) + bqv_ref[...]
    for h in range(H):
        qv_ref[h] = mm[:, 128 * h:128 * (h + 1)]
    hsh = jnp.dot(x, wh_ref[...], preferred_element_type=jnp.float32)
    ang_ref[...] = hsh[:, :H] / (hsh[:, H:] + EPS)


def _projections(x, W_qv, b_qv, Wh2, b):
    B, S, D = x.shape
    Sb = 512
    grid = (S // Sb,)
    return pl.pallas_call(
        _proj_body,
        grid=grid,
        in_specs=[
            pl.BlockSpec((1, Sb, D), lambda s: (b, s, 0)),
            pl.BlockSpec((D, 2 * D), lambda s: (0, 0)),
            pl.BlockSpec((1, 2 * D), lambda s: (0, 0)),
            pl.BlockSpec((D, 2 * H), lambda s: (0, 0)),
        ],
        out_specs=[
            pl.BlockSpec((H, Sb, 128), lambda s: (0, s, 0)),
            pl.BlockSpec((Sb, H), lambda s: (s, 0)),
        ],
        out_shape=[
            jax.ShapeDtypeStruct((H, S, 128), jnp.float32),
            jax.ShapeDtypeStruct((S, H), jnp.float32),
        ],
    )(x, W_qv, b_qv, Wh2)


def _ce_stage(keys, payload, g, d, k, CS=1024):
    bit_d = (g & d) != 0
    swapm = bit_d ^ ((g & k) != 0)
    if d < CS:
        pk = jnp.where(bit_d, jnp.roll(keys, d, axis=0), jnp.roll(keys, -d, axis=0))
        pp = jnp.where(bit_d, jnp.roll(payload, d, axis=0), jnp.roll(payload, -d, axis=0))
    else:
        dl = (d // CS) * 16
        pk = jnp.where(bit_d, jnp.roll(keys, dl, axis=1), jnp.roll(keys, -dl, axis=1))
        pp = jnp.where(bit_d, jnp.roll(payload, dl, axis=1), jnp.roll(payload, -dl, axis=1))
    cmp = (pk < keys) | ((pk == keys) & (pp < payload))
    take = cmp ^ swapm
    return jnp.where(take, pk, keys), jnp.where(take, pp, payload)


def _sort_body(keys_ref, out_ref, *, S):
    keys = keys_ref[...]
    sub = lax.broadcasted_iota(jnp.int32, keys.shape, 0)
    lane = lax.broadcasted_iota(jnp.int32, keys.shape, 1)
    g = (lane >> 4) * 1024 + sub
    payload = g
    kk = 2
    while kk <= S:
        d = kk // 2
        while d >= 1:
            keys, payload = _ce_stage(keys, payload, g, d, kk)
            d //= 2
        kk *= 2
    out_ref[...] = (lane & (H - 1)) * S + payload


def _bitonic_sort(keys, S):
    return pl.pallas_call(
        functools.partial(_sort_body, S=S),
        grid=(1,),
        in_specs=[pl.BlockSpec((1024, 128), lambda i: (0, 0))],
        out_specs=pl.BlockSpec((1024, 128), lambda i: (0, 0)),
        out_shape=jax.ShapeDtypeStruct((1024, 128), jnp.int32),
    )(keys)


def _sc_gather(qv_flat, sidx):
    NP, NJ = sidx.shape[0], sidx.shape[1]
    S = NJ * 128
    info = plsc.get_sparse_core_info()
    NC = info.num_cores
    mesh = plsc.VectorSubcoreMesh(core_axis_name="c", subcore_axis_name="s")

    @functools.partial(
        pl.kernel, mesh=mesh,
        out_type=jax.ShapeDtypeStruct((NP, S, 128), jnp.float32),
        scratch_types=[
            pltpu.VMEM((NJ // 2, 128), jnp.int32),
            pltpu.VMEM((128, 128), jnp.float32),
            pltpu.SemaphoreType.DMA,
        ],
    )
    def k(qv_hbm, sidx_hbm, out_hbm, idx_v, buf, sem):
        wid = lax.axis_index("s") * NC + lax.axis_index("c")
        p = wid >> 1
        jbase = (wid & 1) * (NJ // 2)
        pltpu.sync_copy(sidx_hbm.at[p, pl.ds(jbase * 1, NJ // 2)], idx_v)

        def body(j, carry):
            pltpu.async_copy(qv_hbm.at[idx_v.at[j]], buf, sem).wait()
            pltpu.sync_copy(buf, out_hbm.at[p, pl.ds((jbase + j) * 128, 128)])
            return carry

        lax.fori_loop(0, NJ // 2, body, 0)

    return k(qv_flat, sidx)


def _sc_scatter(os_, sidx):
    NP, NJ = sidx.shape[0], sidx.shape[1]
    S = NJ * 128
    info = plsc.get_sparse_core_info()
    NC = info.num_cores
    mesh = plsc.VectorSubcoreMesh(core_axis_name="c", subcore_axis_name="s")

    @functools.partial(
        pl.kernel, mesh=mesh,
        out_type=jax.ShapeDtypeStruct((NP * S, 128), jnp.float32),
        scratch_types=[
            pltpu.VMEM((NJ // 2, 128), jnp.int32),
            pltpu.VMEM((128, 128), jnp.float32),
            pltpu.SemaphoreType.DMA,
        ],
    )
    def k(os_hbm, sidx_hbm, out_hbm, idx_v, buf, sem):
        wid = lax.axis_index("s") * NC + lax.axis_index("c")
        p = wid >> 1
        jbase = (wid & 1) * (NJ // 2)
        pltpu.sync_copy(sidx_hbm.at[p, pl.ds(jbase * 1, NJ // 2)], idx_v)

        def body(j, carry):
            pltpu.sync_copy(os_hbm.at[p, pl.ds((jbase + j) * 128, 128)], buf)
            pltpu.async_copy(buf, out_hbm.at[idx_v.at[j]], sem).wait()
            return carry

        lax.fori_loop(0, NJ // 2, body, 0)

    return k(os_, sidx)


def _attn_body(qv_ref, o_ref, *, Ts, S):
    qv = qv_ref[0]
    bi = lax.broadcasted_iota(jnp.int32, (Ts, Ts), 0) // BUCKET
    bj = lax.broadcasted_iota(jnp.int32, (Ts, Ts), 1) // BUCKET
    mask01 = jnp.where(bi == bj, 1.0, 0.0)
    for t in range(S // Ts):
        blk = qv[Ts * t:Ts * (t + 1)]
        q = blk[:, :64].astype(jnp.bfloat16)
        v = blk[:, 64:].astype(jnp.bfloat16)
        s = lax.dot_general(q, q, (((1,), (1,)), ((), ())),
                            preferred_element_type=jnp.float32)
        e = jnp.exp(s * 0.125) * mask01
        p = e / jnp.sum(e, axis=-1, keepdims=True)
        o = jnp.dot(p.astype(jnp.bfloat16), v,
                    preferred_element_type=jnp.float32)
        o_ref[0, Ts * t:Ts * (t + 1), :64] = o
        o_ref[0, Ts * t:Ts * (t + 1), 64:] = jnp.zeros_like(o)


def _bucket_attention(qvs):
    NP, S, _ = qvs.shape
    Ts = 256
    grid = (NP,)
    return pl.pallas_call(
        functools.partial(_attn_body, Ts=Ts, S=S),
        grid=grid,
        in_specs=[pl.BlockSpec((1, S, 128), lambda g: (g, 0, 0))],
        out_specs=pl.BlockSpec((1, S, 128), lambda g: (g, 0, 0)),
        out_shape=jax.ShapeDtypeStruct((NP, S, 128), jnp.float32),
    )(qvs)


def _outproj_body(o4_ref, wo_ref, bo_ref, out_ref):
    acc = bo_ref[...].astype(jnp.float32)
    for kgrp in range(4):
        blk4 = jnp.concatenate(
            [o4_ref[4 * kgrp + j, :, :64] for j in range(4)], axis=1)
        acc = acc + jnp.dot(blk4, wo_ref[256 * kgrp:256 * (kgrp + 1), :],
                            preferred_element_type=jnp.float32)
    out_ref[0] = acc


def _outproj_body_alias(o4_ref, wo_ref, bo_ref, prev_ref, out_ref):
    del prev_ref
    _outproj_body(o4_ref, wo_ref, bo_ref, out_ref)


def _out_projection(o4, W_o, b_o, b, B, prev=None):
    _, S, _ = o4.shape
    D = W_o.shape[0]
    Sb = 512
    grid = (S // Sb,)
    in_specs = [
        pl.BlockSpec((H, Sb, 128), lambda s: (0, s, 0)),
        pl.BlockSpec((D, D), lambda s: (0, 0)),
        pl.BlockSpec((1, D), lambda s: (0, 0)),
    ]
    args = [o4, W_o, b_o.reshape(1, D)]
    body = _outproj_body
    kwargs = {}
    if prev is not None:
        in_specs.append(pl.BlockSpec(memory_space=pl.ANY))
        args.append(prev)
        body = _outproj_body_alias
        kwargs = dict(input_output_aliases={3: 0})
    return pl.pallas_call(
        body,
        grid=grid,
        in_specs=in_specs,
        out_specs=pl.BlockSpec((1, Sb, D), lambda s: (b, s, 0)),
        out_shape=jax.ShapeDtypeStruct((B, S, D), jnp.float32),
        **kwargs,
    )(*args)


def kernel(x, W_hash, W_q, b_q, W_v, b_v, W_o, b_o):
    B, S, D = x.shape
    dh = D // H
    W_qv = jnp.concatenate(
        [W_q.reshape(D, H, dh), W_v.reshape(D, H, dh)], axis=2).reshape(D, 2 * D)
    b_qv = jnp.concatenate(
        [b_q.reshape(H, dh), b_v.reshape(H, dh)], axis=1).reshape(1, 2 * D)
    Wh2 = W_hash.reshape(D, H, 2).transpose(0, 2, 1).reshape(D, 2 * H)

    def pack_keys(ang_b):
        return ang_b.reshape(8, 1024, H).transpose(1, 0, 2).reshape(1024, 128)

    def unpack_sidx(sidxp):
        return (sidxp.reshape(1024, 8, H).transpose(2, 1, 0)
                .reshape(H, S // 128, 128))

    qv0, ang0 = _projections(x, W_qv, b_qv, Wh2, 0)
    sidx0 = unpack_sidx(_bitonic_sort(pack_keys(ang0), S))
    qvs0 = _sc_gather(qv0.reshape(H * S, 128), sidx0)
    qv1, ang1 = _projections(x, W_qv, b_qv, Wh2, 1)
    sidx1 = unpack_sidx(_bitonic_sort(pack_keys(ang1), S))
    os0 = _bucket_attention(qvs0)
    qvs1 = _sc_gather(qv1.reshape(H * S, 128), sidx1)
    o40 = _sc_scatter(os0, sidx0).reshape(H, S, 128)
    os1 = _bucket_attention(qvs1)
    out0 = _out_projection(o40, W_o, b_o, 0, B)
    o41 = _sc_scatter(os1, sidx1).reshape(H, S, 128)
    return _out_projection(o41, W_o, b_o, 1, B, prev=out0)
